# baseline (speedup 1.0000x reference)
import jax
import jax.numpy as jnp
from jax.experimental import pallas as pl
from jax.experimental.pallas import tpu as pltpu


def _moe_step(x_ref, sim_ref, thr_ref, w1_hbm, w2_hbm,
              final_ref, scores_ref, eof_hbm, k_ref,
              stag1, stag2, wb1, wb2, eofscr, xbs, sem_w, sem_o):
    e = pl.program_id(0)
    n = pl.program_id(1)
    ne = pl.num_programs(0)
    nb = pl.num_programs(1)
    tb = eofscr.shape[0]
    thr = thr_ref[0, 0]
    step = e * nb + n

    @pl.when(step == 0)
    def _():
        xf = x_ref[...]
        xn = xf / (jnp.sqrt(jnp.sum(xf * xf, axis=1, keepdims=True)) + 1e-12)
        sm = sim_ref[...]
        wn = sm / (jnp.sqrt(jnp.sum(sm * sm, axis=0, keepdims=True)) + 1e-12)
        s = jnp.dot(xn, wn, preferred_element_type=jnp.float32)
        scores_ref[...] = s
        k_ref[...] = jnp.sum((s > thr).astype(jnp.int32), axis=1, keepdims=True)
        xbs[...] = xf.astype(jnp.bfloat16)
        pltpu.make_async_copy(w1_hbm.at[0], stag1, sem_w).start()
        pltpu.make_async_copy(w2_hbm.at[0], stag2, sem_w).start()
        pltpu.make_async_copy(w1_hbm.at[0], stag1, sem_w).wait()
        pltpu.make_async_copy(w2_hbm.at[0], stag2, sem_w).wait()

    @pl.when(n == 0)
    def _():
        slot = jax.lax.rem(e, 2)

        @pl.when(e > 0)
        def _():
            pltpu.make_async_copy(w1_hbm.at[e], stag1, sem_w).wait()
            pltpu.make_async_copy(w2_hbm.at[e], stag2, sem_w).wait()

        wb1[slot] = stag1[...].astype(jnp.bfloat16)
        wb2[slot] = stag2[...].astype(jnp.bfloat16)

        @pl.when(e + 1 < ne)
        def _():
            pltpu.make_async_copy(w1_hbm.at[e + 1], stag1, sem_w).start()
            pltpu.make_async_copy(w2_hbm.at[e + 1], stag2, sem_w).start()

    slot = jax.lax.rem(e, 2)
    rows = pl.ds(n * tb, tb)
    xb = xbs[rows, :]
    h = jax.lax.dot_general(xb, wb1[slot], (((1,), (1,)), ((), ())),
                            preferred_element_type=jnp.float32)
    g = 0.5 * h * (1.0 + jax.lax.erf(h * 0.7071067811865476))
    out = jax.lax.dot_general(g.astype(jnp.bfloat16), wb2[slot],
                              (((1,), (1,)), ((), ())),
                              preferred_element_type=jnp.float32)
    s_blk = scores_ref[rows, :]
    onehot = (jax.lax.broadcasted_iota(jnp.int32, s_blk.shape, 1) == e)
    mask_col = jnp.sum(
        jnp.where((s_blk > thr) & onehot, 1.0, 0.0), axis=1, keepdims=True)
    mo = out * mask_col

    @pl.when(e == 0)
    def _():
        final_ref[rows, :] = mo

    @pl.when(e != 0)
    def _():
        final_ref[rows, :] = final_ref[rows, :] + mo

    @pl.when(step > 0)
    def _():
        sp = step - 1
        ep = sp // nb
        np_ = jax.lax.rem(sp, nb)
        prev_dst = eof_hbm.at[pl.ds(np_ * tb, tb), ep, :]
        pltpu.make_async_copy(eofscr, prev_dst, sem_o).wait()

    eofscr[...] = mo
    dst = eof_hbm.at[rows, e, :]
    pltpu.make_async_copy(eofscr, dst, sem_o).start()

    @pl.when(step == ne * nb - 1)
    def _():
        pltpu.make_async_copy(eofscr, dst, sem_o).wait()


def kernel(hidden_states, sim_matrix, threshold, w1, w2):
    Bb, Tt, Cc = hidden_states.shape
    Ee, Dff, _ = w1.shape
    N = Bb * Tt
    TB = 256
    NB = N // TB

    x32 = hidden_states.reshape(N, Cc)
    thr = threshold.reshape(1, 1)

    grid = (Ee, NB)
    out_shapes = (
        jax.ShapeDtypeStruct((N, Cc), jnp.float32),
        jax.ShapeDtypeStruct((N, Ee), jnp.float32),
        jax.ShapeDtypeStruct((N, Ee, Cc), jnp.float32),
        jax.ShapeDtypeStruct((N, 1), jnp.int32),
    )
    in_specs = [
        pl.BlockSpec((N, Cc), lambda e, n: (0, 0)),
        pl.BlockSpec((Cc, Ee), lambda e, n: (0, 0)),
        pl.BlockSpec((1, 1), lambda e, n: (0, 0)),
        pl.BlockSpec(memory_space=pltpu.MemorySpace.HBM),
        pl.BlockSpec(memory_space=pltpu.MemorySpace.HBM),
    ]
    out_specs = (
        pl.BlockSpec((N, Cc), lambda e, n: (0, 0)),
        pl.BlockSpec((N, Ee), lambda e, n: (0, 0)),
        pl.BlockSpec(memory_space=pltpu.MemorySpace.HBM),
        pl.BlockSpec((N, 1), lambda e, n: (0, 0)),
    )
    scratch_shapes = [
        pltpu.VMEM((Dff, Cc), jnp.float32),
        pltpu.VMEM((Cc, Dff), jnp.float32),
        pltpu.VMEM((2, Dff, Cc), jnp.bfloat16),
        pltpu.VMEM((2, Cc, Dff), jnp.bfloat16),
        pltpu.VMEM((TB, Cc), jnp.float32),
        pltpu.VMEM((N, Cc), jnp.bfloat16),
        pltpu.SemaphoreType.DMA,
        pltpu.SemaphoreType.DMA,
    ]
    final, scores, eof, k = pl.pallas_call(
        _moe_step,
        grid=grid,
        in_specs=in_specs,
        out_specs=out_specs,
        out_shape=out_shapes,
        scratch_shapes=scratch_shapes,
        compiler_params=pltpu.CompilerParams(
            dimension_semantics=("arbitrary", "arbitrary"),
            vmem_limit_bytes=67000000,
        ),
    )(x32, sim_matrix, thr, w1, w2)

    return (final.reshape(Bb, Tt, Cc), scores, eof, k.reshape(N))


# speedup vs baseline: 1.1143x; 1.1143x over previous
import jax
import jax.numpy as jnp
from jax.experimental import pallas as pl
from jax.experimental.pallas import tpu as pltpu


def _moe_step(x_ref, sim_ref, thr_ref, w1_hbm, w2_hbm,
              final_ref, scores_ref, eof_hbm, k_ref,
              stag1, stag2, wb1, wb2, eofscr, xbs, sem_w, sem_o):
    e = pl.program_id(0)
    n = pl.program_id(1)
    ne = pl.num_programs(0)
    nb = pl.num_programs(1)
    tb = eofscr.shape[0]
    thr = thr_ref[0, 0]
    step = e * nb + n

    @pl.when(step == 0)
    def _():
        xf = x_ref[...]
        xn = xf / (jnp.sqrt(jnp.sum(xf * xf, axis=1, keepdims=True)) + 1e-12)
        sm = sim_ref[...]
        wn = sm / (jnp.sqrt(jnp.sum(sm * sm, axis=0, keepdims=True)) + 1e-12)
        s = jnp.dot(xn, wn, preferred_element_type=jnp.float32)
        scores_ref[...] = s
        k_ref[...] = jnp.sum((s > thr).astype(jnp.int32), axis=1, keepdims=True)
        xbs[...] = xf.astype(jnp.bfloat16)
        pltpu.make_async_copy(w1_hbm.at[0], stag1, sem_w).start()
        pltpu.make_async_copy(w2_hbm.at[0], stag2, sem_w).start()
        pltpu.make_async_copy(w1_hbm.at[0], stag1, sem_w).wait()
        pltpu.make_async_copy(w2_hbm.at[0], stag2, sem_w).wait()

    @pl.when(n == 0)
    def _():
        slot = jax.lax.rem(e, 2)

        @pl.when(e > 0)
        def _():
            pltpu.make_async_copy(w1_hbm.at[e], stag1, sem_w).wait()
            pltpu.make_async_copy(w2_hbm.at[e], stag2, sem_w).wait()

        wb1[slot] = stag1[...].astype(jnp.bfloat16)
        wb2[slot] = stag2[...].astype(jnp.bfloat16)

        @pl.when(e + 1 < ne)
        def _():
            pltpu.make_async_copy(w1_hbm.at[e + 1], stag1, sem_w).start()
            pltpu.make_async_copy(w2_hbm.at[e + 1], stag2, sem_w).start()

    slot = jax.lax.rem(e, 2)
    rows = pl.ds(n * tb, tb)
    xb = xbs[rows, :]
    h = jax.lax.dot_general(xb, wb1[slot], (((1,), (1,)), ((), ())),
                            preferred_element_type=jnp.float32)
    g = 0.5 * h * (1.0 + jax.lax.erf(h * 0.7071067811865476))
    out = jax.lax.dot_general(g.astype(jnp.bfloat16), wb2[slot],
                              (((1,), (1,)), ((), ())),
                              preferred_element_type=jnp.float32)
    s_blk = scores_ref[rows, :]
    onehot = (jax.lax.broadcasted_iota(jnp.int32, s_blk.shape, 1) == e)
    mask_col = jnp.sum(
        jnp.where((s_blk > thr) & onehot, 1.0, 0.0), axis=1, keepdims=True)
    mo = out * mask_col

    @pl.when(e == 0)
    def _():
        final_ref[rows, :] = mo

    @pl.when(e != 0)
    def _():
        final_ref[rows, :] = final_ref[rows, :] + mo

    @pl.when(step > 0)
    def _():
        sp = step - 1
        ep = sp // nb
        np_ = jax.lax.rem(sp, nb)
        prev_dst = eof_hbm.at[pl.ds(np_ * tb, tb), ep, :]
        pltpu.make_async_copy(eofscr, prev_dst, sem_o).wait()

    eofscr[...] = mo
    dst = eof_hbm.at[rows, e, :]
    pltpu.make_async_copy(eofscr, dst, sem_o).start()

    @pl.when(step == ne * nb - 1)
    def _():
        pltpu.make_async_copy(eofscr, dst, sem_o).wait()


def kernel(hidden_states, sim_matrix, threshold, w1, w2):
    Bb, Tt, Cc = hidden_states.shape
    Ee, Dff, _ = w1.shape
    N = Bb * Tt
    TB = 512
    NB = N // TB

    x32 = hidden_states.reshape(N, Cc)
    thr = threshold.reshape(1, 1)

    grid = (Ee, NB)
    out_shapes = (
        jax.ShapeDtypeStruct((N, Cc), jnp.float32),
        jax.ShapeDtypeStruct((N, Ee), jnp.float32),
        jax.ShapeDtypeStruct((N, Ee, Cc), jnp.float32),
        jax.ShapeDtypeStruct((N, 1), jnp.int32),
    )
    in_specs = [
        pl.BlockSpec((N, Cc), lambda e, n: (0, 0)),
        pl.BlockSpec((Cc, Ee), lambda e, n: (0, 0)),
        pl.BlockSpec((1, 1), lambda e, n: (0, 0)),
        pl.BlockSpec(memory_space=pltpu.MemorySpace.HBM),
        pl.BlockSpec(memory_space=pltpu.MemorySpace.HBM),
    ]
    out_specs = (
        pl.BlockSpec((N, Cc), lambda e, n: (0, 0)),
        pl.BlockSpec((N, Ee), lambda e, n: (0, 0)),
        pl.BlockSpec(memory_space=pltpu.MemorySpace.HBM),
        pl.BlockSpec((N, 1), lambda e, n: (0, 0)),
    )
    scratch_shapes = [
        pltpu.VMEM((Dff, Cc), jnp.float32),
        pltpu.VMEM((Cc, Dff), jnp.float32),
        pltpu.VMEM((2, Dff, Cc), jnp.bfloat16),
        pltpu.VMEM((2, Cc, Dff), jnp.bfloat16),
        pltpu.VMEM((TB, Cc), jnp.float32),
        pltpu.VMEM((N, Cc), jnp.bfloat16),
        pltpu.SemaphoreType.DMA,
        pltpu.SemaphoreType.DMA,
    ]
    final, scores, eof, k = pl.pallas_call(
        _moe_step,
        grid=grid,
        in_specs=in_specs,
        out_specs=out_specs,
        out_shape=out_shapes,
        scratch_shapes=scratch_shapes,
        compiler_params=pltpu.CompilerParams(
            dimension_semantics=("arbitrary", "arbitrary"),
            vmem_limit_bytes=67000000,
        ),
    )(x32, sim_matrix, thr, w1, w2)

    return (final.reshape(Bb, Tt, Cc), scores, eof, k.reshape(N))
